# baseline (speedup 1.0000x reference)
import jax
import jax.numpy as jnp
from jax.experimental import pallas as pl
from jax.experimental.pallas import tpu as pltpu

B, H, W = 16, 384, 384
CI = 1
NCH = B // CI
DEPTH = 8
CR = CI * H


def _loss_kernel(label_ref, x_hbm, p_hbm, n_hbm, out_ref,
                 xb, pb, nb, acc_ref, cnt_ref, sems):

    def start(k):
        slot = k % DEPTH
        rows = pl.ds(k * CR, CR)
        pltpu.make_async_copy(x_hbm.at[rows, :], xb.at[slot], sems.at[slot, 0]).start()
        pltpu.make_async_copy(p_hbm.at[rows, :], pb.at[slot], sems.at[slot, 1]).start()
        pltpu.make_async_copy(n_hbm.at[rows, :], nb.at[slot], sems.at[slot, 2]).start()

    def wait(k):
        slot = k % DEPTH
        rows = pl.ds(k * CR, CR)
        pltpu.make_async_copy(x_hbm.at[rows, :], xb.at[slot], sems.at[slot, 0]).wait()
        pltpu.make_async_copy(p_hbm.at[rows, :], pb.at[slot], sems.at[slot, 1]).wait()
        pltpu.make_async_copy(n_hbm.at[rows, :], nb.at[slot], sems.at[slot, 2]).wait()

    for k in range(DEPTH):
        start(k)

    total = 0.0
    count = 0.0
    for k in range(NCH):
        slot = k % DEPTH
        wait(k)
        for j in range(CI):
            x = xb[slot, pl.ds(j * H, H), :]
            p = pb[slot, pl.ds(j * H, H), :]
            n = nb[slot, pl.ds(j * H, H), :]
            m = jnp.logical_and(p > 0.5, n > 0.5).astype(jnp.float32)
            y = label_ref[k * CI + j].astype(jnp.float32)
            u = jnp.exp(-jnp.abs(x))
            t = 1.0 + u * m
            v = t
            for _ in range(6):
                half = v.shape[0] // 2
                v = v[:half] * v[half:]
            total += ((1.0 + y) * (jnp.sum(m * jnp.maximum(x, 0.0))
                                   + jnp.sum(jnp.log(v)))
                      - (2.0 * y) * jnp.sum(m * x))
            count += jnp.sum(m)
        if k + DEPTH < NCH:
            start(k + DEPTH)
    acc_ref[0] = total
    cnt_ref[0] = count
    out_ref[0] = total / count


def kernel(cancer_logits, prostate_mask, needle_mask, label, involvement):
    x = cancer_logits.reshape(B * H, W)
    p = prostate_mask.reshape(B * H, W)
    n = needle_mask.reshape(B * H, W)
    grid_spec = pltpu.PrefetchScalarGridSpec(
        num_scalar_prefetch=1,
        grid=(1,),
        in_specs=[
            pl.BlockSpec(memory_space=pl.ANY),
            pl.BlockSpec(memory_space=pl.ANY),
            pl.BlockSpec(memory_space=pl.ANY),
        ],
        out_specs=pl.BlockSpec(memory_space=pltpu.SMEM),
        scratch_shapes=[
            pltpu.VMEM((DEPTH, CR, W), jnp.float32),
            pltpu.VMEM((DEPTH, CR, W), jnp.float32),
            pltpu.VMEM((DEPTH, CR, W), jnp.float32),
            pltpu.SMEM((1,), jnp.float32),
            pltpu.SMEM((1,), jnp.float32),
            pltpu.SemaphoreType.DMA((DEPTH, 3)),
        ],
    )
    out = pl.pallas_call(
        _loss_kernel,
        grid_spec=grid_spec,
        out_shape=jax.ShapeDtypeStruct((1,), jnp.float32),
    )(label.astype(jnp.int32), x, p, n)
    return out[0]


# speedup vs baseline: 1.0049x; 1.0049x over previous
import jax
import jax.numpy as jnp
from jax.experimental import pallas as pl
from jax.experimental.pallas import tpu as pltpu

B, H, W = 16, 384, 384
CI = 2
NCH = B // CI
DEPTH = 6
CR = CI * H


def _loss_kernel(label_ref, x_hbm, p_hbm, n_hbm, out_ref,
                 xb, pb, nb, acc_ref, cnt_ref, sems):

    def start(k):
        slot = k % DEPTH
        rows = pl.ds(k * CR, CR)
        pltpu.make_async_copy(x_hbm.at[rows, :], xb.at[slot], sems.at[slot, 0]).start()
        pltpu.make_async_copy(p_hbm.at[rows, :], pb.at[slot], sems.at[slot, 1]).start()
        pltpu.make_async_copy(n_hbm.at[rows, :], nb.at[slot], sems.at[slot, 2]).start()

    def wait(k):
        slot = k % DEPTH
        rows = pl.ds(k * CR, CR)
        pltpu.make_async_copy(x_hbm.at[rows, :], xb.at[slot], sems.at[slot, 0]).wait()
        pltpu.make_async_copy(p_hbm.at[rows, :], pb.at[slot], sems.at[slot, 1]).wait()
        pltpu.make_async_copy(n_hbm.at[rows, :], nb.at[slot], sems.at[slot, 2]).wait()

    for k in range(DEPTH):
        start(k)

    total = 0.0
    count = 0.0
    for k in range(NCH):
        slot = k % DEPTH
        wait(k)
        for j in range(CI):
            x = xb[slot, pl.ds(j * H, H), :]
            p = pb[slot, pl.ds(j * H, H), :]
            n = nb[slot, pl.ds(j * H, H), :]
            m = jnp.logical_and(p > 0.5, n > 0.5).astype(jnp.float32)
            y = label_ref[k * CI + j].astype(jnp.float32)
            u = jnp.exp(-jnp.abs(x))
            t = 1.0 + u * m
            v = t
            for _ in range(6):
                half = v.shape[0] // 2
                v = v[:half] * v[half:]
            total += ((1.0 + y) * (jnp.sum(m * jnp.maximum(x, 0.0))
                                   + jnp.sum(jnp.log(v)))
                      - (2.0 * y) * jnp.sum(m * x))
            count += jnp.sum(m)
        if k + DEPTH < NCH:
            start(k + DEPTH)
    acc_ref[0] = total
    cnt_ref[0] = count
    out_ref[0] = total / count


def kernel(cancer_logits, prostate_mask, needle_mask, label, involvement):
    x = cancer_logits.reshape(B * H, W)
    p = prostate_mask.reshape(B * H, W)
    n = needle_mask.reshape(B * H, W)
    grid_spec = pltpu.PrefetchScalarGridSpec(
        num_scalar_prefetch=1,
        grid=(1,),
        in_specs=[
            pl.BlockSpec(memory_space=pl.ANY),
            pl.BlockSpec(memory_space=pl.ANY),
            pl.BlockSpec(memory_space=pl.ANY),
        ],
        out_specs=pl.BlockSpec(memory_space=pltpu.SMEM),
        scratch_shapes=[
            pltpu.VMEM((DEPTH, CR, W), jnp.float32),
            pltpu.VMEM((DEPTH, CR, W), jnp.float32),
            pltpu.VMEM((DEPTH, CR, W), jnp.float32),
            pltpu.SMEM((1,), jnp.float32),
            pltpu.SMEM((1,), jnp.float32),
            pltpu.SemaphoreType.DMA((DEPTH, 3)),
        ],
    )
    out = pl.pallas_call(
        _loss_kernel,
        grid_spec=grid_spec,
        out_shape=jax.ShapeDtypeStruct((1,), jnp.float32),
    )(label.astype(jnp.int32), x, p, n)
    return out[0]


# speedup vs baseline: 1.0510x; 1.0458x over previous
import jax
import jax.numpy as jnp
from jax.experimental import pallas as pl
from jax.experimental.pallas import tpu as pltpu

B, H, W = 16, 384, 384
HALF = H // 2
CHUNKS = (4, 4, 4, 4, 4, 4, 4, 2, 1, 1)
DEPTH = 4
MAXC = max(CHUNKS)
_BASES = []
_acc = 0
for _c in CHUNKS:
    _BASES.append(_acc)
    _acc += _c
assert _acc == 2 * B


def _loss_kernel(label_ref, x_hbm, p_hbm, n_hbm, out_ref,
                 xb, pb, nb, acc_ref, cnt_ref, sems):

    def copies(idx):
        slot = idx % DEPTH
        nr = CHUNKS[idx] * HALF
        rows = pl.ds(_BASES[idx] * HALF, nr)
        dst = pl.ds(0, nr)
        return (
            pltpu.make_async_copy(x_hbm.at[rows, :], xb.at[slot, dst, :], sems.at[slot, 0]),
            pltpu.make_async_copy(p_hbm.at[rows, :], pb.at[slot, dst, :], sems.at[slot, 1]),
            pltpu.make_async_copy(n_hbm.at[rows, :], nb.at[slot, dst, :], sems.at[slot, 2]),
        )

    for idx in range(DEPTH):
        for cp in copies(idx):
            cp.start()

    total = 0.0
    count = 0.0
    for idx in range(len(CHUNKS)):
        slot = idx % DEPTH
        for cp in copies(idx):
            cp.wait()
        for h in range(CHUNKS[idx]):
            rows = pl.ds(h * HALF, HALF)
            x = xb[slot, rows, :]
            p = pb[slot, rows, :]
            n = nb[slot, rows, :]
            m = jnp.logical_and(p > 0.5, n > 0.5).astype(jnp.float32)
            y = label_ref[(_BASES[idx] + h) // 2].astype(jnp.float32)
            u = jnp.exp(-jnp.abs(x))
            t = 1.0 + u * m
            v = t
            for _ in range(5):
                half = v.shape[0] // 2
                v = v[:half] * v[half:]
            total += ((1.0 + y) * (jnp.sum(m * jnp.maximum(x, 0.0))
                                   + jnp.sum(jnp.log(v)))
                      - (2.0 * y) * jnp.sum(m * x))
            count += jnp.sum(m)
        if idx + DEPTH < len(CHUNKS):
            for cp in copies(idx + DEPTH):
                cp.start()
    acc_ref[0] = total
    cnt_ref[0] = count
    out_ref[0] = total / count


def kernel(cancer_logits, prostate_mask, needle_mask, label, involvement):
    x = cancer_logits.reshape(B * H, W)
    p = prostate_mask.reshape(B * H, W)
    n = needle_mask.reshape(B * H, W)
    grid_spec = pltpu.PrefetchScalarGridSpec(
        num_scalar_prefetch=1,
        grid=(1,),
        in_specs=[
            pl.BlockSpec(memory_space=pl.ANY),
            pl.BlockSpec(memory_space=pl.ANY),
            pl.BlockSpec(memory_space=pl.ANY),
        ],
        out_specs=pl.BlockSpec(memory_space=pltpu.SMEM),
        scratch_shapes=[
            pltpu.VMEM((DEPTH, MAXC * HALF, W), jnp.float32),
            pltpu.VMEM((DEPTH, MAXC * HALF, W), jnp.float32),
            pltpu.VMEM((DEPTH, MAXC * HALF, W), jnp.float32),
            pltpu.SMEM((1,), jnp.float32),
            pltpu.SMEM((1,), jnp.float32),
            pltpu.SemaphoreType.DMA((DEPTH, 3)),
        ],
    )
    out = pl.pallas_call(
        _loss_kernel,
        grid_spec=grid_spec,
        out_shape=jax.ShapeDtypeStruct((1,), jnp.float32),
    )(label.astype(jnp.int32), x, p, n)
    return out[0]


# speedup vs baseline: 1.0959x; 1.0427x over previous
import jax
import jax.numpy as jnp
from jax.experimental import pallas as pl
from jax.experimental.pallas import tpu as pltpu

B, H, W = 16, 384, 384
HALF = H // 4
CHUNKS = (8, 8, 8, 8, 8, 8, 8, 4, 2, 1, 1)
DEPTH = 4
MAXC = max(CHUNKS)
_BASES = []
_acc = 0
for _c in CHUNKS:
    _BASES.append(_acc)
    _acc += _c
assert _acc == 4 * B


def _loss_kernel(label_ref, x_hbm, p_hbm, n_hbm, out_ref,
                 xb, pb, nb, acc_ref, cnt_ref, sems):

    def copies(idx):
        slot = idx % DEPTH
        nr = CHUNKS[idx] * HALF
        rows = pl.ds(_BASES[idx] * HALF, nr)
        dst = pl.ds(0, nr)
        return (
            pltpu.make_async_copy(x_hbm.at[rows, :], xb.at[slot, dst, :], sems.at[slot, 0]),
            pltpu.make_async_copy(p_hbm.at[rows, :], pb.at[slot, dst, :], sems.at[slot, 1]),
            pltpu.make_async_copy(n_hbm.at[rows, :], nb.at[slot, dst, :], sems.at[slot, 2]),
        )

    for idx in range(DEPTH):
        for cp in copies(idx):
            cp.start()

    total = 0.0
    count = 0.0
    for idx in range(len(CHUNKS)):
        slot = idx % DEPTH
        for cp in copies(idx):
            cp.wait()
        for h in range(CHUNKS[idx]):
            rows = pl.ds(h * HALF, HALF)
            x = xb[slot, rows, :]
            p = pb[slot, rows, :]
            n = nb[slot, rows, :]
            m = jnp.logical_and(p > 0.5, n > 0.5).astype(jnp.float32)
            y = label_ref[(_BASES[idx] + h) // 4].astype(jnp.float32)
            u = jnp.exp(-jnp.abs(x))
            t = 1.0 + u * m
            v = t
            for _ in range(5):
                half = v.shape[0] // 2
                v = v[:half] * v[half:]
            total += ((1.0 + y) * (jnp.sum(m * jnp.maximum(x, 0.0))
                                   + jnp.sum(jnp.log(v)))
                      - (2.0 * y) * jnp.sum(m * x))
            count += jnp.sum(m)
        if idx + DEPTH < len(CHUNKS):
            for cp in copies(idx + DEPTH):
                cp.start()
    acc_ref[0] = total
    cnt_ref[0] = count
    out_ref[0] = total / count


def kernel(cancer_logits, prostate_mask, needle_mask, label, involvement):
    x = cancer_logits.reshape(B * H, W)
    p = prostate_mask.reshape(B * H, W)
    n = needle_mask.reshape(B * H, W)
    grid_spec = pltpu.PrefetchScalarGridSpec(
        num_scalar_prefetch=1,
        grid=(1,),
        in_specs=[
            pl.BlockSpec(memory_space=pl.ANY),
            pl.BlockSpec(memory_space=pl.ANY),
            pl.BlockSpec(memory_space=pl.ANY),
        ],
        out_specs=pl.BlockSpec(memory_space=pltpu.SMEM),
        scratch_shapes=[
            pltpu.VMEM((DEPTH, MAXC * HALF, W), jnp.float32),
            pltpu.VMEM((DEPTH, MAXC * HALF, W), jnp.float32),
            pltpu.VMEM((DEPTH, MAXC * HALF, W), jnp.float32),
            pltpu.SMEM((1,), jnp.float32),
            pltpu.SMEM((1,), jnp.float32),
            pltpu.SemaphoreType.DMA((DEPTH, 3)),
        ],
    )
    out = pl.pallas_call(
        _loss_kernel,
        grid_spec=grid_spec,
        out_shape=jax.ShapeDtypeStruct((1,), jnp.float32),
    )(label.astype(jnp.int32), x, p, n)
    return out[0]
